# baseline (speedup 1.0000x reference)
import functools
import numpy as np
import jax
import jax.numpy as jnp
from jax import lax
from jax.experimental import pallas as pl
from jax.experimental.pallas import tpu as pltpu
from jax.experimental.pallas import tpu_sc as plsc

_K = 64
_B = 4
_S = 4096
_SHARDS = 8
_CHUNK = _S // _SHARDS
_NV = _CHUNK // 16
_I32MIN = np.int32(-2**31)


def _splat(x, dtype=None):
    x = jnp.asarray(x) if dtype is None else jnp.asarray(x, dtype)
    return jnp.broadcast_to(x, (16,))


def _mod(x, m):
    c = np.float32(1.0 / m)
    q = (x.astype(jnp.float32) * c).astype(jnp.int32)
    r = x - q * np.int32(m)
    r = r + jnp.where(r < 0, np.int32(m), np.int32(0))
    r = r - jnp.where(r >= m, np.int32(m), np.int32(0))
    return r


def _body(ids_hbm, par_hbm, gain_hbm, mu_hbm, sal_hbm,
          ids_v, par_v, comb_v, uk_v, hist_v, hrd_v, sv_v, srd_v,
          sal_v, gain_v, mu16_v, hist_sh, stats_sh):
    c = lax.axis_index("c")
    s = lax.axis_index("s")
    lr = s // 8
    j = s % 8
    r = c * 2 + lr
    base = r * _S + j * _CHUNK

    pltpu.sync_copy(par_hbm, par_v)
    pltpu.sync_copy(ids_hbm.at[pl.ds(base, _CHUNK)], ids_v)

    iota = lax.iota(jnp.int32, 16)
    one_v = plsc.load_gather(par_v, [jnp.full((16,), 48, jnp.int32)])
    zero_v = plsc.load_gather(par_v, [jnp.full((16,), 49, jnp.int32)])

    for i in range(_NV):
        ids = ids_v[pl.ds(i * 16, 16)]
        amp = plsc.load_gather(par_v, [_mod(ids, 17)])
        pit = plsc.load_gather(par_v, [_mod(ids, 31) + 17])
        bnd = jnp.where(_mod(ids, 7) == 0, one_v, zero_v)
        comb = (amp + pit) + bnd
        comb_v[pl.ds(i * 16, 16)] = comb
        u = plsc.bitcast(comb, jnp.int32)
        uk = jnp.where(u < 0, jnp.bitwise_xor(u, np.int32(-1)),
                       jnp.bitwise_xor(u, _I32MIN))
        uk_v[pl.ds(i * 16, 16)] = uk

    ones16 = jnp.ones((16,), jnp.int32)
    prefix = jnp.int32(0)
    kk = jnp.int32(_K)
    hi_masks = (np.int32(-(2**8)), np.int32(-(2**16)), np.int32(-(2**24)),
                np.int32(0))
    for m in (3, 2, 1, 0):
        for t in range(16):
            hist_v[pl.ds(t * 16, 16)] = jnp.zeros((16,), jnp.int32)
        hm = _splat(hi_masks[m])
        pf = _splat(prefix)
        for i in range(_NV):
            uk = uk_v[pl.ds(i * 16, 16)]
            surv = (uk & hm) == pf
            d = lax.shift_right_logical(uk, np.int32(8 * m)) & np.int32(255)
            plsc.addupdate_scatter(hist_v, [d], ones16, mask=surv)
        off_w = ((m * 2 + lr) * 8 + j) * 256
        pltpu.sync_copy(hist_v, hist_sh.at[pl.ds(off_w, 256)])
        plsc.subcore_barrier()
        pltpu.sync_copy(hist_sh.at[pl.ds((m * 2 + lr) * 2048, 2048)], hrd_v)

        running = jnp.int32(0)
        dstar = jnp.int32(-1)
        sstar = jnp.int32(-1)
        kkv = _splat(kk)
        for t in range(15, -1, -1):
            cnt = hrd_v[pl.ds(t * 16, 16)]
            for sh in range(1, 8):
                cnt = cnt + hrd_v[pl.ds(sh * 256 + t * 16, 16)]
            suf = lax.rev(plsc.cumsum(lax.rev(cnt, (0,))), (0,))
            sg = (suf - cnt) + _splat(running)
            found = (sg < kkv) & (sg + cnt >= kkv)
            dst_c = jnp.max(jnp.where(found, iota + np.int32(t * 16),
                                      np.int32(-1)))
            ss_c = jnp.max(jnp.where(found, sg, np.int32(-1)))
            dstar = jnp.maximum(dstar, dst_c)
            sstar = jnp.maximum(sstar, ss_c)
            running = running + jnp.sum(cnt)
        prefix = prefix | lax.shift_left(dstar, np.int32(8 * m))
        kk = kk - sstar

    t_v = _splat(prefix)
    st_v = t_v ^ _I32MIN

    acc_eq = jnp.zeros((16,), jnp.int32)
    acc_gt = jnp.zeros((16,), jnp.float32)
    for i in range(_NV):
        uk = uk_v[pl.ds(i * 16, 16)]
        comb = comb_v[pl.ds(i * 16, 16)]
        su = uk ^ _I32MIN
        acc_eq = acc_eq + jnp.where(uk == t_v, np.int32(1), np.int32(0))
        acc_gt = acc_gt + jnp.where(su > st_v, comb, np.float32(0.0))
    local_eq = jnp.sum(acc_eq)
    local_gts = jnp.sum(acc_gt)
    local_eq_f = local_eq.astype(jnp.float32)
    sv = jnp.where(iota == _splat(j), _splat(local_eq_f), np.float32(0.0))
    sv = sv + jnp.where(iota == _splat(j + 8), _splat(local_gts),
                        np.float32(0.0))
    sv_v[...] = sv
    pltpu.sync_copy(sv_v, stats_sh.at[pl.ds((lr * 8 + j) * 16, 16)])
    plsc.subcore_barrier()
    pltpu.sync_copy(stats_sh.at[pl.ds(lr * 128, 128)], srd_v)
    comb8 = srd_v[pl.ds(0, 16)]
    for sh in range(1, 8):
        comb8 = comb8 + srd_v[pl.ds(sh * 16, 16)]
    eqvec = jnp.where(iota < 8, comb8, np.float32(0.0))
    ecs = plsc.cumsum(eqvec)
    eq_incl = jnp.sum(jnp.where(iota == _splat(j), ecs, np.float32(0.0)))
    eq_before = (eq_incl - local_eq_f).astype(jnp.int32)
    gts_tot = jnp.sum(jnp.where(iota >= 8, comb8, np.float32(0.0)))
    take = jnp.maximum(jnp.int32(0), jnp.minimum(kk - eq_before, local_eq))

    sk = prefix ^ _I32MIN
    ut = jnp.where(sk >= 0, sk, sk ^ np.int32(0x7FFFFFFF))
    vt_v = plsc.bitcast(_splat(ut), jnp.float32)
    mu_v = (_splat(gts_tot) + kk.astype(jnp.float32) * vt_v) \
        * np.float32(1.0 / 64.0)

    take_v = _splat(take)
    run = jnp.int32(0)
    for i in range(_NV):
        uk = uk_v[pl.ds(i * 16, 16)]
        comb = comb_v[pl.ds(i * 16, 16)]
        su = uk ^ _I32MIN
        eq = uk == t_v
        eqi = jnp.where(eq, np.int32(1), np.int32(0))
        excl = (plsc.cumsum(eqi) - eqi) + _splat(run)
        keep = (su > st_v) | (eq & (excl < take_v))
        sal = jnp.where(keep, comb, np.float32(0.0))
        sal_v[pl.ds(i * 16, 16)] = sal
        gain_v[pl.ds(i * 16, 16)] = mu_v * (np.float32(1.0) + sal)
        run = run + jnp.sum(eqi)
    pltpu.sync_copy(sal_v, sal_hbm.at[pl.ds(base, _CHUNK)])
    pltpu.sync_copy(gain_v, gain_hbm.at[pl.ds(base, _CHUNK)])

    @pl.when(j == 0)
    def _():
        mu16_v[...] = mu_v
        pltpu.sync_copy(mu16_v, mu_hbm.at[pl.ds(r * 16, 16)])


@jax.jit
def _run(ids_flat, params):
    mesh = plsc.VectorSubcoreMesh(core_axis_name="c", subcore_axis_name="s",
                                  num_cores=2, num_subcores=16)
    f = functools.partial(
        pl.kernel,
        out_type=(
            jax.ShapeDtypeStruct((_B * _S,), jnp.float32),
            jax.ShapeDtypeStruct((_B * 16,), jnp.float32),
            jax.ShapeDtypeStruct((_B * _S,), jnp.float32),
        ),
        mesh=mesh,
        compiler_params=pltpu.CompilerParams(needs_layout_passes=False),
        scratch_types=[
            pltpu.VMEM((_CHUNK,), jnp.int32),
            pltpu.VMEM((64,), jnp.float32),
            pltpu.VMEM((_CHUNK,), jnp.float32),
            pltpu.VMEM((_CHUNK,), jnp.int32),
            pltpu.VMEM((256,), jnp.int32),
            pltpu.VMEM((2048,), jnp.int32),
            pltpu.VMEM((16,), jnp.float32),
            pltpu.VMEM((128,), jnp.float32),
            pltpu.VMEM((_CHUNK,), jnp.float32),
            pltpu.VMEM((_CHUNK,), jnp.float32),
            pltpu.VMEM((16,), jnp.float32),
            pltpu.VMEM_SHARED((4 * 2 * 8 * 256,), jnp.int32),
            pltpu.VMEM_SHARED((256,), jnp.float32),
        ],
    )(_body)
    return f(ids_flat, params)


def kernel(input_ids, channel_w):
    ids_flat = input_ids.reshape(-1)
    r17 = jnp.arange(17, dtype=jnp.float32) / jnp.float32(17.0)
    r31 = jnp.arange(31, dtype=jnp.float32) / jnp.float32(31.0)
    params = jnp.concatenate([
        channel_w[0] * r17,
        channel_w[1] * r31,
        jnp.stack([channel_w[2] * jnp.float32(1.0),
                   channel_w[2] * jnp.float32(0.0)]),
        jnp.zeros((14,), jnp.float32),
    ])
    gains_f, mu_pad, sal_f = _run(ids_flat, params)
    gains = gains_f.reshape(_B, _S)
    salience = sal_f.reshape(_B, _S)
    mu_scalar = mu_pad.reshape(_B, 16)[:, 0]
    return (gains, mu_scalar, salience)


# speedup vs baseline: 1.0490x; 1.0490x over previous
import functools
import numpy as np
import jax
import jax.numpy as jnp
from jax import lax
from jax.experimental import pallas as pl
from jax.experimental.pallas import tpu as pltpu
from jax.experimental.pallas import tpu_sc as plsc

_K = 64
_B = 4
_S = 4096
_CHUNK = _S // 8
_NV = _CHUNK // 16
_I32MIN = np.int32(-2**31)

_BASE = np.zeros(64, np.float32)
_BASE[0:17] = np.arange(17, dtype=np.float32) / np.float32(17.0)
_BASE[17:48] = np.arange(31, dtype=np.float32) / np.float32(31.0)
_BASE[48] = 1.0
_BASE[49] = 0.0
_WSEL = np.zeros(64, np.int32)
_WSEL[0:17] = 0
_WSEL[17:48] = 1
_WSEL[48:] = 2


def _splat(x, dtype=None):
    x = jnp.asarray(x) if dtype is None else jnp.asarray(x, dtype)
    return jnp.broadcast_to(x, (16,))


def _mod(x, m):
    c = np.float32(1.0 / m)
    q = (x.astype(jnp.float32) * c).astype(jnp.int32)
    r = x - q * np.int32(m)
    r = r + jnp.where(r < 0, np.int32(m), np.int32(0))
    r = r - jnp.where(r >= m, np.int32(m), np.int32(0))
    return r


def _body(ids_hbm, chw_hbm, base_hbm, gain_hbm, mu_hbm, sal_hbm,
          ids_v, chw_v, base_v, tab_v, comb_v, uk_v, hist_v, hrd_v, sv_v,
          srd_v, sal_v, gain_v, mu16_v, hist_sh, stats_sh):
    c = lax.axis_index("c")
    s = lax.axis_index("s")
    lr = s // 8
    j = s % 8
    r = c * 2 + lr
    col = j * _CHUNK

    pltpu.sync_copy(chw_hbm, chw_v)
    pltpu.sync_copy(base_hbm, base_v)
    pltpu.sync_copy(ids_hbm.at[r, pl.ds(col, _CHUNK)], ids_v)

    iota = lax.iota(jnp.int32, 16)
    w0 = plsc.load_gather(chw_v, [jnp.zeros((16,), jnp.int32)])
    w1 = plsc.load_gather(chw_v, [jnp.ones((16,), jnp.int32)])
    w2 = plsc.load_gather(chw_v, [jnp.full((16,), 2, jnp.int32)])
    for q in range(4):
        g = iota + np.int32(q * 16)
        w = jnp.where(g < 17, w0, jnp.where(g < 48, w1, w2))
        tab_v[pl.ds(q * 16, 16)] = w * base_v[pl.ds(q * 16, 16)]
    one_v = plsc.load_gather(tab_v, [jnp.full((16,), 48, jnp.int32)])
    zero_v = plsc.load_gather(tab_v, [jnp.full((16,), 49, jnp.int32)])

    for i in range(_NV):
        ids = ids_v[pl.ds(i * 16, 16)]
        amp = plsc.load_gather(tab_v, [_mod(ids, 17)])
        pit = plsc.load_gather(tab_v, [_mod(ids, 31) + 17])
        bnd = jnp.where(_mod(ids, 7) == 0, one_v, zero_v)
        comb = (amp + pit) + bnd
        comb_v[pl.ds(i * 16, 16)] = comb
        u = plsc.bitcast(comb, jnp.int32)
        uk = jnp.where(u < 0, jnp.bitwise_xor(u, np.int32(-1)),
                       jnp.bitwise_xor(u, _I32MIN))
        uk_v[pl.ds(i * 16, 16)] = uk

    ones16 = jnp.ones((16,), jnp.int32)
    prefix = jnp.int32(0)
    kk = jnp.int32(_K)
    hi_masks = (np.int32(-(2**8)), np.int32(-(2**16)), np.int32(-(2**24)),
                np.int32(0))
    for m in (3, 2, 1, 0):
        for t in range(16):
            hist_v[pl.ds(t * 16, 16)] = jnp.zeros((16,), jnp.int32)
        hm = _splat(hi_masks[m])
        pf = _splat(prefix)
        for i in range(_NV):
            uk = uk_v[pl.ds(i * 16, 16)]
            surv = (uk & hm) == pf
            d = lax.shift_right_logical(uk, np.int32(8 * m)) & np.int32(255)
            plsc.addupdate_scatter(hist_v, [d], ones16, mask=surv)
        off_w = ((m * 2 + lr) * 8 + j) * 256
        pltpu.sync_copy(hist_v, hist_sh.at[pl.ds(off_w, 256)])
        plsc.subcore_barrier()
        pltpu.sync_copy(hist_sh.at[pl.ds((m * 2 + lr) * 2048, 2048)], hrd_v)

        running = jnp.int32(0)
        dstar = jnp.int32(-1)
        sstar = jnp.int32(-1)
        kkv = _splat(kk)
        for t in range(15, -1, -1):
            cnt = hrd_v[pl.ds(t * 16, 16)]
            for sh in range(1, 8):
                cnt = cnt + hrd_v[pl.ds(sh * 256 + t * 16, 16)]
            suf = lax.rev(plsc.cumsum(lax.rev(cnt, (0,))), (0,))
            sg = (suf - cnt) + _splat(running)
            found = (sg < kkv) & (sg + cnt >= kkv)
            dst_c = jnp.max(jnp.where(found, iota + np.int32(t * 16),
                                      np.int32(-1)))
            ss_c = jnp.max(jnp.where(found, sg, np.int32(-1)))
            dstar = jnp.maximum(dstar, dst_c)
            sstar = jnp.maximum(sstar, ss_c)
            running = running + jnp.sum(cnt)
        prefix = prefix | lax.shift_left(dstar, np.int32(8 * m))
        kk = kk - sstar
    d0 = dstar

    t_v = _splat(prefix)
    st_v = t_v ^ _I32MIN

    gidx = jnp.where(iota < 8, iota, np.int32(0)) * 256 + _splat(d0)
    eqv = plsc.load_gather(hrd_v, [gidx], mask=iota < 8)
    jv = _splat(j)
    local_eq = jnp.sum(jnp.where(iota == jv, eqv, np.int32(0)))
    eq_before = jnp.sum(jnp.where((iota < jv) & (iota < 8), eqv, np.int32(0)))
    take = jnp.maximum(jnp.int32(0), jnp.minimum(kk - eq_before, local_eq))

    acc_gt = jnp.zeros((16,), jnp.float32)
    for i in range(_NV):
        uk = uk_v[pl.ds(i * 16, 16)]
        su = uk ^ _I32MIN
        acc_gt = acc_gt + jnp.where(su > st_v, comb_v[pl.ds(i * 16, 16)],
                                    np.float32(0.0))
    local_gts = jnp.sum(acc_gt)
    sv_v[...] = jnp.where(iota == jv, _splat(local_gts), np.float32(0.0))
    pltpu.sync_copy(sv_v, stats_sh.at[pl.ds((lr * 8 + j) * 16, 16)])
    plsc.subcore_barrier()
    pltpu.sync_copy(stats_sh.at[pl.ds(lr * 128, 128)], srd_v)
    comb8 = srd_v[pl.ds(0, 16)]
    for sh in range(1, 8):
        comb8 = comb8 + srd_v[pl.ds(sh * 16, 16)]
    gts_tot = jnp.sum(jnp.where(iota < 8, comb8, np.float32(0.0)))

    sk = prefix ^ _I32MIN
    ut = jnp.where(sk >= 0, sk, sk ^ np.int32(0x7FFFFFFF))
    vt_v = plsc.bitcast(_splat(ut), jnp.float32)
    mu_v = (_splat(gts_tot) + kk.astype(jnp.float32) * vt_v) \
        * np.float32(1.0 / 64.0)

    take_v = _splat(take)
    run = jnp.int32(0)
    for i in range(_NV):
        uk = uk_v[pl.ds(i * 16, 16)]
        comb = comb_v[pl.ds(i * 16, 16)]
        su = uk ^ _I32MIN
        eq = uk == t_v
        eqi = jnp.where(eq, np.int32(1), np.int32(0))
        excl = (plsc.cumsum(eqi) - eqi) + _splat(run)
        keep = (su > st_v) | (eq & (excl < take_v))
        sal = jnp.where(keep, comb, np.float32(0.0))
        sal_v[pl.ds(i * 16, 16)] = sal
        gain_v[pl.ds(i * 16, 16)] = mu_v * (np.float32(1.0) + sal)
        run = run + jnp.sum(eqi)
    pltpu.sync_copy(sal_v, sal_hbm.at[r, pl.ds(col, _CHUNK)])
    pltpu.sync_copy(gain_v, gain_hbm.at[r, pl.ds(col, _CHUNK)])

    @pl.when(j == 0)
    def _():
        mu16_v[...] = mu_v
        pltpu.sync_copy(mu16_v, mu_hbm.at[r])


@jax.jit
def _run(input_ids, channel_w):
    mesh = plsc.VectorSubcoreMesh(core_axis_name="c", subcore_axis_name="s",
                                  num_cores=2, num_subcores=16)
    f = functools.partial(
        pl.kernel,
        out_type=(
            jax.ShapeDtypeStruct((_B, _S), jnp.float32),
            jax.ShapeDtypeStruct((_B, 16), jnp.float32),
            jax.ShapeDtypeStruct((_B, _S), jnp.float32),
        ),
        mesh=mesh,
        compiler_params=pltpu.CompilerParams(needs_layout_passes=False),
        scratch_types=[
            pltpu.VMEM((_CHUNK,), jnp.int32),
            pltpu.VMEM((3,), jnp.float32),
            pltpu.VMEM((64,), jnp.float32),
            pltpu.VMEM((64,), jnp.float32),
            pltpu.VMEM((_CHUNK,), jnp.float32),
            pltpu.VMEM((_CHUNK,), jnp.int32),
            pltpu.VMEM((256,), jnp.int32),
            pltpu.VMEM((2048,), jnp.int32),
            pltpu.VMEM((16,), jnp.float32),
            pltpu.VMEM((128,), jnp.float32),
            pltpu.VMEM((_CHUNK,), jnp.float32),
            pltpu.VMEM((_CHUNK,), jnp.float32),
            pltpu.VMEM((16,), jnp.float32),
            pltpu.VMEM_SHARED((4 * 2 * 8 * 256,), jnp.int32),
            pltpu.VMEM_SHARED((256,), jnp.float32),
        ],
    )(_body)
    return f(input_ids, channel_w, jnp.asarray(_BASE))


def kernel(input_ids, channel_w):
    gains, mu_pad, salience = _run(input_ids, channel_w)
    return (gains, mu_pad[:, 0], salience)


# speedup vs baseline: 1.0524x; 1.0033x over previous
import functools
import numpy as np
import jax
import jax.numpy as jnp
from jax import lax
from jax.experimental import pallas as pl
from jax.experimental.pallas import tpu as pltpu
from jax.experimental.pallas import tpu_sc as plsc

_K = 64
_B = 4
_S = 4096
_CHUNK = _S // 8
_NV = _CHUNK // 16
_I32MIN = np.int32(-2**31)

_BASE = np.zeros(64, np.float32)
_BASE[0:17] = np.arange(17, dtype=np.float32) / np.float32(17.0)
_BASE[17:48] = np.arange(31, dtype=np.float32) / np.float32(31.0)
_BASE[48] = 1.0
_BASE[49] = 0.0
_WSEL = np.zeros(64, np.int32)
_WSEL[0:17] = 0
_WSEL[17:48] = 1
_WSEL[48:] = 2


def _splat(x, dtype=None):
    x = jnp.asarray(x) if dtype is None else jnp.asarray(x, dtype)
    return jnp.broadcast_to(x, (16,))


def _mod(x, m):
    c = np.float32(1.0 / m)
    q = (x.astype(jnp.float32) * c).astype(jnp.int32)
    r = x - q * np.int32(m)
    r = r + jnp.where(r < 0, np.int32(m), np.int32(0))
    r = r - jnp.where(r >= m, np.int32(m), np.int32(0))
    return r


def _body(ids_hbm, chw_hbm, base_hbm, gain_hbm, mu_hbm, sal_hbm,
          ids_v, chw_v, base_v, tab_v, comb_v, uk_v, hist_v, hrd_v, sv_v,
          srd_v, sal_v, gain_v, mu16_v, hist_sh, stats_sh):
    c = lax.axis_index("c")
    s = lax.axis_index("s")
    lr = s // 8
    j = s % 8
    r = c * 2 + lr
    col = j * _CHUNK

    pltpu.sync_copy(chw_hbm, chw_v.at[pl.ds(0, 3)])
    pltpu.sync_copy(base_hbm, base_v)
    pltpu.sync_copy(ids_hbm.at[r, pl.ds(col, _CHUNK)], ids_v)

    iota = lax.iota(jnp.int32, 16)
    chwvec = chw_v[pl.ds(0, 16)]
    w0 = jnp.broadcast_to(chwvec[0], (16,))
    w1 = jnp.broadcast_to(chwvec[1], (16,))
    w2 = jnp.broadcast_to(chwvec[2], (16,))
    for q in range(4):
        g = iota + np.int32(q * 16)
        w = jnp.where(g < 17, w0, jnp.where(g < 48, w1, w2))
        tab_v[pl.ds(q * 16, 16)] = w * base_v[pl.ds(q * 16, 16)]
    tail = tab_v[pl.ds(48, 16)]
    one_v = jnp.broadcast_to(tail[0], (16,))
    zero_v = jnp.broadcast_to(tail[1], (16,))

    for i in range(_NV):
        ids = ids_v[pl.ds(i * 16, 16)]
        amp = plsc.load_gather(tab_v, [_mod(ids, 17)])
        pit = plsc.load_gather(tab_v, [_mod(ids, 31) + 17])
        bnd = jnp.where(_mod(ids, 7) == 0, one_v, zero_v)
        comb = (amp + pit) + bnd
        comb_v[pl.ds(i * 16, 16)] = comb
        u = plsc.bitcast(comb, jnp.int32)
        uk = jnp.where(u < 0, jnp.bitwise_xor(u, np.int32(-1)),
                       jnp.bitwise_xor(u, _I32MIN))
        uk_v[pl.ds(i * 16, 16)] = uk

    ones16 = jnp.ones((16,), jnp.int32)
    prefix = jnp.int32(0)
    kk = jnp.int32(_K)
    hi_masks = (np.int32(-(2**8)), np.int32(-(2**16)), np.int32(-(2**24)),
                np.int32(0))
    for m in (3, 2, 1, 0):
        for t in range(16):
            hist_v[pl.ds(t * 16, 16)] = jnp.zeros((16,), jnp.int32)
        hm = _splat(hi_masks[m])
        pf = _splat(prefix)
        for i in range(_NV):
            uk = uk_v[pl.ds(i * 16, 16)]
            surv = (uk & hm) == pf
            d = lax.shift_right_logical(uk, np.int32(8 * m)) & np.int32(255)
            plsc.addupdate_scatter(hist_v, [d], ones16, mask=surv)
        off_w = ((m * 2 + lr) * 8 + j) * 256
        pltpu.sync_copy(hist_v, hist_sh.at[pl.ds(off_w, 256)])
        plsc.subcore_barrier()
        pltpu.sync_copy(hist_sh.at[pl.ds((m * 2 + lr) * 2048, 2048)], hrd_v)

        running = jnp.int32(0)
        dstar = jnp.int32(-1)
        sstar = jnp.int32(-1)
        kkv = _splat(kk)
        for t in range(15, -1, -1):
            cnt = hrd_v[pl.ds(t * 16, 16)]
            for sh in range(1, 8):
                cnt = cnt + hrd_v[pl.ds(sh * 256 + t * 16, 16)]
            suf = lax.rev(plsc.cumsum(lax.rev(cnt, (0,))), (0,))
            sg = (suf - cnt) + _splat(running)
            found = (sg < kkv) & (sg + cnt >= kkv)
            dst_c = jnp.max(jnp.where(found, iota + np.int32(t * 16),
                                      np.int32(-1)))
            ss_c = jnp.max(jnp.where(found, sg, np.int32(-1)))
            dstar = jnp.maximum(dstar, dst_c)
            sstar = jnp.maximum(sstar, ss_c)
            running = running + jnp.sum(cnt)
        prefix = prefix | lax.shift_left(dstar, np.int32(8 * m))
        kk = kk - sstar
    d0 = dstar

    t_v = _splat(prefix)
    st_v = t_v ^ _I32MIN

    gidx = jnp.where(iota < 8, iota, np.int32(0)) * 256 + _splat(d0)
    eqv = plsc.load_gather(hrd_v, [gidx], mask=iota < 8)
    jv = _splat(j)
    local_eq = jnp.sum(jnp.where(iota == jv, eqv, np.int32(0)))
    eq_before = jnp.sum(jnp.where((iota < jv) & (iota < 8), eqv, np.int32(0)))
    take = jnp.maximum(jnp.int32(0), jnp.minimum(kk - eq_before, local_eq))

    acc_gt = jnp.zeros((16,), jnp.float32)
    for i in range(_NV):
        uk = uk_v[pl.ds(i * 16, 16)]
        su = uk ^ _I32MIN
        acc_gt = acc_gt + jnp.where(su > st_v, comb_v[pl.ds(i * 16, 16)],
                                    np.float32(0.0))
    local_gts = jnp.sum(acc_gt)
    sv_v[...] = jnp.where(iota == jv, _splat(local_gts), np.float32(0.0))
    pltpu.sync_copy(sv_v, stats_sh.at[pl.ds((lr * 8 + j) * 16, 16)])
    plsc.subcore_barrier()
    pltpu.sync_copy(stats_sh.at[pl.ds(lr * 128, 128)], srd_v)
    comb8 = srd_v[pl.ds(0, 16)]
    for sh in range(1, 8):
        comb8 = comb8 + srd_v[pl.ds(sh * 16, 16)]
    gts_tot = jnp.sum(jnp.where(iota < 8, comb8, np.float32(0.0)))

    sk = prefix ^ _I32MIN
    ut = jnp.where(sk >= 0, sk, sk ^ np.int32(0x7FFFFFFF))
    vt_v = plsc.bitcast(_splat(ut), jnp.float32)
    mu_v = (_splat(gts_tot) + kk.astype(jnp.float32) * vt_v) \
        * np.float32(1.0 / 64.0)

    take_v = _splat(take)
    run = jnp.int32(0)
    for i in range(_NV):
        uk = uk_v[pl.ds(i * 16, 16)]
        comb = comb_v[pl.ds(i * 16, 16)]
        su = uk ^ _I32MIN
        eq = uk == t_v
        eqi = jnp.where(eq, np.int32(1), np.int32(0))
        excl = (plsc.cumsum(eqi) - eqi) + _splat(run)
        keep = (su > st_v) | (eq & (excl < take_v))
        sal = jnp.where(keep, comb, np.float32(0.0))
        sal_v[pl.ds(i * 16, 16)] = sal
        gain_v[pl.ds(i * 16, 16)] = mu_v * (np.float32(1.0) + sal)
        run = run + jnp.sum(eqi)
    pltpu.sync_copy(sal_v, sal_hbm.at[r, pl.ds(col, _CHUNK)])
    pltpu.sync_copy(gain_v, gain_hbm.at[r, pl.ds(col, _CHUNK)])

    @pl.when(j == 0)
    def _():
        mu16_v[...] = mu_v
        pltpu.sync_copy(mu16_v, mu_hbm.at[r])


@jax.jit
def _run(input_ids, channel_w):
    mesh = plsc.VectorSubcoreMesh(core_axis_name="c", subcore_axis_name="s",
                                  num_cores=2, num_subcores=16)
    f = functools.partial(
        pl.kernel,
        out_type=(
            jax.ShapeDtypeStruct((_B, _S), jnp.float32),
            jax.ShapeDtypeStruct((_B, 16), jnp.float32),
            jax.ShapeDtypeStruct((_B, _S), jnp.float32),
        ),
        mesh=mesh,
        compiler_params=pltpu.CompilerParams(needs_layout_passes=False),
        scratch_types=[
            pltpu.VMEM((_CHUNK,), jnp.int32),
            pltpu.VMEM((16,), jnp.float32),
            pltpu.VMEM((64,), jnp.float32),
            pltpu.VMEM((64,), jnp.float32),
            pltpu.VMEM((_CHUNK,), jnp.float32),
            pltpu.VMEM((_CHUNK,), jnp.int32),
            pltpu.VMEM((256,), jnp.int32),
            pltpu.VMEM((2048,), jnp.int32),
            pltpu.VMEM((16,), jnp.float32),
            pltpu.VMEM((128,), jnp.float32),
            pltpu.VMEM((_CHUNK,), jnp.float32),
            pltpu.VMEM((_CHUNK,), jnp.float32),
            pltpu.VMEM((16,), jnp.float32),
            pltpu.VMEM_SHARED((4 * 2 * 8 * 256,), jnp.int32),
            pltpu.VMEM_SHARED((256,), jnp.float32),
        ],
    )(_body)
    return f(input_ids, channel_w, jnp.asarray(_BASE))


def kernel(input_ids, channel_w):
    gains, mu_pad, salience = _run(input_ids, channel_w)
    return (gains, mu_pad[:, 0], salience)


# speedup vs baseline: 1.2049x; 1.1448x over previous
import functools
import numpy as np
import jax
import jax.numpy as jnp
from jax import lax
from jax.experimental import pallas as pl
from jax.experimental.pallas import tpu as pltpu
from jax.experimental.pallas import tpu_sc as plsc

_K = 64
_B = 4
_S = 4096
_CHUNK = _S // 8
_NV = _CHUNK // 16
_I32MIN = np.int32(-2**31)

_BASE = np.zeros(64, np.float32)
_BASE[0:17] = np.arange(17, dtype=np.float32) / np.float32(17.0)
_BASE[17:48] = np.arange(31, dtype=np.float32) / np.float32(31.0)
_BASE[48] = 1.0
_BASE[49] = 0.0


def _splat(x, dtype=None):
    x = jnp.asarray(x) if dtype is None else jnp.asarray(x, dtype)
    return jnp.broadcast_to(x, (16,))


def _mod(x, m):
    c = np.float32(1.0 / m)
    q = (x.astype(jnp.float32) * c).astype(jnp.int32)
    r = x - q * np.int32(m)
    r = r + jnp.where(r < 0, np.int32(m), np.int32(0))
    r = r - jnp.where(r >= m, np.int32(m), np.int32(0))
    return r


def _body(ids_hbm, chw_hbm, base_hbm, gain_hbm, mu_hbm, sal_hbm,
          ids_v, chw_v, base_v, tab_v, comb_v, uk_v, hist_v, hrd_v, sv_v,
          srd_v, sal_v, gain_v, mu16_v, hist_sh, stats_sh):
    c = lax.axis_index("c")
    s = lax.axis_index("s")
    lr = s // 8
    j = s % 8
    r = c * 2 + lr
    col = j * _CHUNK

    pltpu.sync_copy(chw_hbm, chw_v.at[pl.ds(0, 3)])
    pltpu.sync_copy(base_hbm, base_v)
    pltpu.sync_copy(ids_hbm.at[r, pl.ds(col, _CHUNK)], ids_v)

    iota = lax.iota(jnp.int32, 16)
    chwvec = chw_v[pl.ds(0, 16)]
    w0 = _splat(chwvec[0])
    w1 = _splat(chwvec[1])
    w2 = _splat(chwvec[2])
    for q in range(4):
        g = iota + np.int32(q * 16)
        w = jnp.where(g < 17, w0, jnp.where(g < 48, w1, w2))
        tab_v[pl.ds(q * 16, 16)] = w * base_v[pl.ds(q * 16, 16)]
    tail = tab_v[pl.ds(48, 16)]
    one_v = _splat(tail[0])
    zero_v = _splat(tail[1])

    def p1(k, carry):
        for u in range(4):
            off = k * 64 + u * 16
            ids = ids_v[pl.ds(off, 16)]
            amp = plsc.load_gather(tab_v, [_mod(ids, 17)])
            pit = plsc.load_gather(tab_v, [_mod(ids, 31) + 17])
            bnd = jnp.where(_mod(ids, 7) == 0, one_v, zero_v)
            comb = (amp + pit) + bnd
            comb_v[pl.ds(off, 16)] = comb
            u32 = plsc.bitcast(comb, jnp.int32)
            uk = jnp.where(u32 < 0, jnp.bitwise_xor(u32, np.int32(-1)),
                           jnp.bitwise_xor(u32, _I32MIN))
            uk_v[pl.ds(off, 16)] = uk
        return carry
    lax.fori_loop(0, _NV // 4, p1, jnp.int32(0))

    ones16 = jnp.ones((16,), jnp.int32)
    zeros16 = jnp.zeros((16,), jnp.int32)
    prefix = jnp.int32(0)
    kk = jnp.int32(_K)
    hi_masks = (np.int32(-(2**8)), np.int32(-(2**16)), np.int32(-(2**24)),
                np.int32(0))
    for m in (3, 2, 1, 0):
        for t in range(16):
            hist_v[pl.ds(t * 16, 16)] = zeros16
        hm = _splat(hi_masks[m])
        pf = _splat(prefix)

        def p2(k, carry):
            for u in range(4):
                off = k * 64 + u * 16
                uk = uk_v[pl.ds(off, 16)]
                surv = (uk & hm) == pf
                d = lax.shift_right_logical(uk, np.int32(8 * m)) \
                    & np.int32(255)
                plsc.addupdate_scatter(hist_v, [d], ones16, mask=surv)
            return carry
        lax.fori_loop(0, _NV // 4, p2, jnp.int32(0))

        off_w = ((m * 2 + lr) * 8 + j) * 256
        pltpu.sync_copy(hist_v, hist_sh.at[pl.ds(off_w, 256)])
        plsc.subcore_barrier()
        pltpu.sync_copy(hist_sh.at[pl.ds((m * 2 + lr) * 2048, 2048)], hrd_v)

        kkv = _splat(kk)

        def pscan(tt, carry):
            running, dstar, sstar = carry
            for u in range(4):
                t16 = (15 - (tt * 4 + u)) * 16
                cnt = hrd_v[pl.ds(t16, 16)]
                for sh in range(1, 8):
                    cnt = cnt + hrd_v[pl.ds(sh * 256 + t16, 16)]
                suf = lax.rev(plsc.cumsum(lax.rev(cnt, (0,))), (0,))
                sg = (suf - cnt) + _splat(running)
                found = (sg < kkv) & (sg + cnt >= kkv)
                dst_c = jnp.max(jnp.where(found, iota + _splat(t16),
                                          np.int32(-1)))
                ss_c = jnp.max(jnp.where(found, sg, np.int32(-1)))
                dstar = jnp.maximum(dstar, dst_c)
                sstar = jnp.maximum(sstar, ss_c)
                running = running + jnp.sum(cnt)
            return running, dstar, sstar
        _, dstar, sstar = lax.fori_loop(
            0, 4, pscan, (jnp.int32(0), jnp.int32(-1), jnp.int32(-1)))
        prefix = prefix | lax.shift_left(dstar, np.int32(8 * m))
        kk = kk - sstar
    d0 = dstar

    t_v = _splat(prefix)
    st_v = t_v ^ _I32MIN

    gidx = jnp.where(iota < 8, iota, np.int32(0)) * 256 + _splat(d0)
    eqv = plsc.load_gather(hrd_v, [gidx], mask=iota < 8)
    jv = _splat(j)
    local_eq = jnp.sum(jnp.where(iota == jv, eqv, np.int32(0)))
    eq_before = jnp.sum(jnp.where((iota < jv) & (iota < 8), eqv, np.int32(0)))
    take = jnp.maximum(jnp.int32(0), jnp.minimum(kk - eq_before, local_eq))

    def p3(k, acc):
        for u in range(4):
            off = k * 64 + u * 16
            uk = uk_v[pl.ds(off, 16)]
            su = uk ^ _I32MIN
            acc = acc + jnp.where(su > st_v, comb_v[pl.ds(off, 16)],
                                  np.float32(0.0))
        return acc
    acc_gt = lax.fori_loop(0, _NV // 4, p3, jnp.zeros((16,), jnp.float32))
    local_gts = jnp.sum(acc_gt)
    sv_v[...] = jnp.where(iota == jv, _splat(local_gts), np.float32(0.0))
    pltpu.sync_copy(sv_v, stats_sh.at[pl.ds((lr * 8 + j) * 16, 16)])
    plsc.subcore_barrier()
    pltpu.sync_copy(stats_sh.at[pl.ds(lr * 128, 128)], srd_v)
    comb8 = srd_v[pl.ds(0, 16)]
    for sh in range(1, 8):
        comb8 = comb8 + srd_v[pl.ds(sh * 16, 16)]
    gts_tot = jnp.sum(jnp.where(iota < 8, comb8, np.float32(0.0)))

    sk = prefix ^ _I32MIN
    ut = jnp.where(sk >= 0, sk, sk ^ np.int32(0x7FFFFFFF))
    vt_v = plsc.bitcast(_splat(ut), jnp.float32)
    mu_v = (_splat(gts_tot) + kk.astype(jnp.float32) * vt_v) \
        * np.float32(1.0 / 64.0)

    take_v = _splat(take)

    def p4(k, run):
        for u in range(4):
            off = k * 64 + u * 16
            uk = uk_v[pl.ds(off, 16)]
            comb = comb_v[pl.ds(off, 16)]
            su = uk ^ _I32MIN
            eq = uk == t_v
            eqi = jnp.where(eq, np.int32(1), np.int32(0))
            excl = (plsc.cumsum(eqi) - eqi) + _splat(run)
            keep = (su > st_v) | (eq & (excl < take_v))
            sal = jnp.where(keep, comb, np.float32(0.0))
            sal_v[pl.ds(off, 16)] = sal
            gain_v[pl.ds(off, 16)] = mu_v * (np.float32(1.0) + sal)
            run = run + jnp.sum(eqi)
        return run
    lax.fori_loop(0, _NV // 4, p4, jnp.int32(0))

    pltpu.sync_copy(sal_v, sal_hbm.at[r, pl.ds(col, _CHUNK)])
    pltpu.sync_copy(gain_v, gain_hbm.at[r, pl.ds(col, _CHUNK)])

    @pl.when(j == 0)
    def _():
        mu16_v[...] = mu_v
        pltpu.sync_copy(mu16_v, mu_hbm.at[r])


@jax.jit
def _run(input_ids, channel_w):
    mesh = plsc.VectorSubcoreMesh(core_axis_name="c", subcore_axis_name="s",
                                  num_cores=2, num_subcores=16)
    f = functools.partial(
        pl.kernel,
        out_type=(
            jax.ShapeDtypeStruct((_B, _S), jnp.float32),
            jax.ShapeDtypeStruct((_B, 16), jnp.float32),
            jax.ShapeDtypeStruct((_B, _S), jnp.float32),
        ),
        mesh=mesh,
        compiler_params=pltpu.CompilerParams(needs_layout_passes=False),
        scratch_types=[
            pltpu.VMEM((_CHUNK,), jnp.int32),
            pltpu.VMEM((16,), jnp.float32),
            pltpu.VMEM((64,), jnp.float32),
            pltpu.VMEM((64,), jnp.float32),
            pltpu.VMEM((_CHUNK,), jnp.float32),
            pltpu.VMEM((_CHUNK,), jnp.int32),
            pltpu.VMEM((256,), jnp.int32),
            pltpu.VMEM((2048,), jnp.int32),
            pltpu.VMEM((16,), jnp.float32),
            pltpu.VMEM((128,), jnp.float32),
            pltpu.VMEM((_CHUNK,), jnp.float32),
            pltpu.VMEM((_CHUNK,), jnp.float32),
            pltpu.VMEM((16,), jnp.float32),
            pltpu.VMEM_SHARED((4 * 2 * 8 * 256,), jnp.int32),
            pltpu.VMEM_SHARED((256,), jnp.float32),
        ],
    )(_body)
    return f(input_ids, channel_w, jnp.asarray(_BASE))


def kernel(input_ids, channel_w):
    gains, mu_pad, salience = _run(input_ids, channel_w)
    return (gains, mu_pad[:, 0], salience)
